# baseline (speedup 1.0000x reference)
import functools

import jax
import jax.numpy as jnp
from jax import lax
from jax.experimental import pallas as pl
from jax.experimental.pallas import tpu as pltpu
from jax.experimental.pallas import tpu_sc as plsc

BATCH = 16384
NUM_FEAT = 32
L = 16
PACK = 128 // NUM_FEAT

_info = plsc.get_sparse_core_info()
NC, NS = _info.num_cores, _info.num_subcores
NW = NC * NS
B_PER_W = BATCH // NW
CHUNK = 128
NCHUNK = B_PER_W // CHUNK
PASS_ELEMS = 256
NPASS = B_PER_W // PASS_ELEMS
CPP = PASS_ELEMS // CHUNK


def _body(uidx_hbm, iidx_hbm, wU_hbm, wI_hbm, out_hbm,
          uraw_v, iraw_v, ushift_v, ishift_v, urows_v, irows_v, out_v, sem):
    wid = lax.axis_index("s") * NC + lax.axis_index("c")
    base = wid * B_PER_W

    pltpu.sync_copy(uidx_hbm.at[pl.ds(base, B_PER_W)], uraw_v)
    pltpu.sync_copy(iidx_hbm.at[pl.ds(base, B_PER_W)], iraw_v)

    for k in range(NCHUNK):
        for t in range(CHUNK // L):
            o = k * CHUNK + t * L
            ushift_v[k, pl.ds(t * L, L)] = uraw_v[pl.ds(o, L)] >> 2
            ishift_v[k, pl.ds(t * L, L)] = iraw_v[pl.ds(o, L)] >> 2

    for p in range(NPASS):
        copies = []
        for c in range(CPP):
            k = p * CPP + c
            copies.append(pltpu.async_copy(
                wU_hbm.at[ushift_v.at[k]],
                urows_v.at[pl.ds(c * CHUNK, CHUNK)], sem))
            copies.append(pltpu.async_copy(
                wI_hbm.at[ishift_v.at[k]],
                irows_v.at[pl.ds(c * CHUNK, CHUNK)], sem))
        for c in copies:
            c.wait()

        def group(g, _):
            row_ids = g * L + lax.iota(jnp.int32, L)
            u16 = uraw_v[pl.ds(p * PASS_ELEMS + g * L, L)]
            i16 = iraw_v[pl.ds(p * PASS_ELEMS + g * L, L)]
            uq = (u16 & (PACK - 1)) * NUM_FEAT
            iq = (i16 & (PACK - 1)) * NUM_FEAT
            acc = jnp.zeros((L,), jnp.float32)
            for j in range(NUM_FEAT):
                u = plsc.load_gather(urows_v, [row_ids, uq + j])
                v = plsc.load_gather(irows_v, [row_ids, iq + j])
                acc = acc + u * v
            pr = 1.0 / (1.0 + jnp.exp(-acc))
            plsc.store_scatter(
                out_v, [p * PASS_ELEMS + g * L + lax.iota(jnp.int32, L)], pr)
            return 0

        lax.fori_loop(0, PASS_ELEMS // L, group, 0)

    pltpu.sync_copy(out_v, out_hbm.at[pl.ds(base, B_PER_W)])


@functools.cache
def _build():
    mesh = plsc.VectorSubcoreMesh(core_axis_name="c", subcore_axis_name="s")
    return pl.kernel(
        _body,
        mesh=mesh,
        compiler_params=pltpu.CompilerParams(use_tc_tiling_on_sc=False,
                                             needs_layout_passes=False),
        out_type=jax.ShapeDtypeStruct((BATCH,), jnp.float32),
        scratch_types=[
            pltpu.VMEM((B_PER_W,), jnp.int32),
            pltpu.VMEM((B_PER_W,), jnp.int32),
            pltpu.VMEM((NCHUNK, CHUNK), jnp.int32),
            pltpu.VMEM((NCHUNK, CHUNK), jnp.int32),
            pltpu.VMEM((PASS_ELEMS, 4 * NUM_FEAT), jnp.float32),
            pltpu.VMEM((PASS_ELEMS, 4 * NUM_FEAT), jnp.float32),
            pltpu.VMEM((B_PER_W,), jnp.float32),
            pltpu.SemaphoreType.DMA,
        ],
    )


def kernel(user_indices, item_indices, w_User, w_Item):
    return _build()(user_indices.astype(jnp.int32),
                    item_indices.astype(jnp.int32),
                    w_User.reshape(-1, 4 * NUM_FEAT),
                    w_Item.reshape(-1, 4 * NUM_FEAT))


# speedup vs baseline: 1.0021x; 1.0021x over previous
import functools

import jax
import jax.numpy as jnp
from jax import lax
from jax.experimental import pallas as pl
from jax.experimental.pallas import tpu as pltpu
from jax.experimental.pallas import tpu_sc as plsc

BATCH = 16384
NUM_FEAT = 32
L = 16
PACK = 128 // NUM_FEAT

_info = plsc.get_sparse_core_info()
NC, NS = _info.num_cores, _info.num_subcores
NW = NC * NS
B_PER_W = BATCH // NW
CHUNK = 128
NCHUNK = B_PER_W // CHUNK
PASS_ELEMS = 256
NPASS = B_PER_W // PASS_ELEMS
CPP = PASS_ELEMS // CHUNK


def _body(uidx_hbm, iidx_hbm, wU_hbm, wI_hbm, out_hbm,
          uraw_v, iraw_v, ushift_v, ishift_v, urows_v, irows_v, out_v, sem):
    wid = lax.axis_index("s") * NC + lax.axis_index("c")
    base = wid * B_PER_W

    pltpu.sync_copy(uidx_hbm.at[pl.ds(base, B_PER_W)], uraw_v)
    pltpu.sync_copy(iidx_hbm.at[pl.ds(base, B_PER_W)], iraw_v)

    for k in range(NCHUNK):
        for t in range(CHUNK // L):
            o = k * CHUNK + t * L
            ushift_v[k, pl.ds(t * L, L)] = uraw_v[pl.ds(o, L)] >> 2
            ishift_v[k, pl.ds(t * L, L)] = iraw_v[pl.ds(o, L)] >> 2

    for p in range(NPASS):
        copies = []
        for c in range(CPP):
            k = p * CPP + c
            copies.append(pltpu.async_copy(
                wU_hbm.at[ushift_v.at[k]],
                urows_v.at[pl.ds(c * CHUNK, CHUNK)], sem))
            copies.append(pltpu.async_copy(
                wI_hbm.at[ishift_v.at[k]],
                irows_v.at[pl.ds(c * CHUNK, CHUNK)], sem))
        for c in copies:
            c.wait()

        def group(g, _):
            row_ids = g * L + lax.iota(jnp.int32, L)
            u16 = uraw_v[pl.ds(p * PASS_ELEMS + g * L, L)]
            i16 = iraw_v[pl.ds(p * PASS_ELEMS + g * L, L)]
            uq = (u16 & (PACK - 1)) * NUM_FEAT
            iq = (i16 & (PACK - 1)) * NUM_FEAT
            acc = jnp.zeros((L,), jnp.float32)
            for j in range(NUM_FEAT):
                u = plsc.load_gather(urows_v, [row_ids, uq + j])
                v = plsc.load_gather(irows_v, [row_ids, iq + j])
                acc = acc + u * v
            pr = 1.0 / (1.0 + jnp.exp(-acc))
            plsc.store_scatter(
                out_v, [p * PASS_ELEMS + g * L + lax.iota(jnp.int32, L)], pr)
            return 0

        lax.fori_loop(0, PASS_ELEMS // L, group, 0)

    pltpu.sync_copy(out_v, out_hbm.at[pl.ds(base, B_PER_W)])


@functools.cache
def _build():
    mesh = plsc.VectorSubcoreMesh(core_axis_name="c", subcore_axis_name="s")
    return pl.kernel(
        _body,
        mesh=mesh,
        compiler_params=pltpu.CompilerParams(use_tc_tiling_on_sc=False,
                                             needs_layout_passes=False),
        out_type=jax.ShapeDtypeStruct((BATCH,), jnp.float32),
        scratch_types=[
            pltpu.VMEM((B_PER_W,), jnp.int32),
            pltpu.VMEM((B_PER_W,), jnp.int32),
            pltpu.VMEM((NCHUNK, CHUNK), jnp.int32),
            pltpu.VMEM((NCHUNK, CHUNK), jnp.int32),
            pltpu.VMEM((PASS_ELEMS, 4 * NUM_FEAT), jnp.float32),
            pltpu.VMEM((PASS_ELEMS, 4 * NUM_FEAT), jnp.float32),
            pltpu.VMEM((B_PER_W,), jnp.float32),
            pltpu.SemaphoreType.DMA,
        ],
    )


def _detile_body(in_ref, out_ref):
    x = in_ref[...]
    xt = jnp.transpose(x)
    xr = xt.reshape(-1, PACK, NUM_FEAT)
    for q in range(PACK):
        out_ref[:, q * NUM_FEAT:(q + 1) * NUM_FEAT] = xr[:, q, :]


@functools.cache
def _build_detile(n_rows):
    bc = 2048
    grid = (n_rows + bc - 1) // bc
    return pl.pallas_call(
        _detile_body,
        grid=(grid,),
        in_specs=[pl.BlockSpec((NUM_FEAT, bc), lambda k: (0, k))],
        out_specs=pl.BlockSpec((bc // PACK, PACK * NUM_FEAT),
                               lambda k: (k, 0)),
        out_shape=jax.ShapeDtypeStruct(
            (n_rows // PACK, PACK * NUM_FEAT), jnp.float32),
    )


def kernel(user_indices, item_indices, w_User, w_Item):
    wU2 = _build_detile(w_User.shape[0])(w_User.T)
    wI2 = _build_detile(w_Item.shape[0])(w_Item.T)
    return _build()(user_indices.astype(jnp.int32),
                    item_indices.astype(jnp.int32),
                    wU2, wI2)


# speedup vs baseline: 1.4012x; 1.3982x over previous
import functools

import jax
import jax.numpy as jnp
from jax import lax
from jax.experimental import pallas as pl
from jax.experimental.pallas import tpu as pltpu
from jax.experimental.pallas import tpu_sc as plsc

BATCH = 16384
NUM_FEAT = 32
L = 16
PACK = 128 // NUM_FEAT

_info = plsc.get_sparse_core_info()
NC, NS = _info.num_cores, _info.num_subcores
NW = NC * NS
B_PER_W = BATCH // NW
CHUNK = 128
NCHUNK = B_PER_W // CHUNK
PASS_ELEMS = 256
NPASS = B_PER_W // PASS_ELEMS
CPP = PASS_ELEMS // CHUNK


def _body(uidx_hbm, iidx_hbm, wU_hbm, wI_hbm, out_hbm,
          uraw_v, iraw_v, ushift_v, ishift_v, urows_v, irows_v, out_v, sem):
    wid = lax.axis_index("s") * NC + lax.axis_index("c")
    base = wid * B_PER_W

    pltpu.sync_copy(uidx_hbm.at[pl.ds(base, B_PER_W)], uraw_v)
    pltpu.sync_copy(iidx_hbm.at[pl.ds(base, B_PER_W)], iraw_v)

    for k in range(NCHUNK):
        for t in range(CHUNK // L):
            o = k * CHUNK + t * L
            u = uraw_v[pl.ds(o, L)]
            i = iraw_v[pl.ds(o, L)]
            ushift_v[k, pl.ds(t * L, L)] = ((u >> 11) << 9) + (u & (SUB - 1))
            ishift_v[k, pl.ds(t * L, L)] = ((i >> 11) << 9) + (i & (SUB - 1))

    for p in range(NPASS):
        copies = []
        for c in range(CPP):
            k = p * CPP + c
            copies.append(pltpu.async_copy(
                wU_hbm.at[ushift_v.at[k]],
                urows_v.at[pl.ds(c * CHUNK, CHUNK)], sem))
            copies.append(pltpu.async_copy(
                wI_hbm.at[ishift_v.at[k]],
                irows_v.at[pl.ds(c * CHUNK, CHUNK)], sem))
        for c in copies:
            c.wait()

        def group(g, _):
            row_ids = g * L + lax.iota(jnp.int32, L)
            u16 = uraw_v[pl.ds(p * PASS_ELEMS + g * L, L)]
            i16 = iraw_v[pl.ds(p * PASS_ELEMS + g * L, L)]
            uq = ((u16 >> 9) & (PACK - 1)) * NUM_FEAT
            iq = ((i16 >> 9) & (PACK - 1)) * NUM_FEAT
            acc = jnp.zeros((L,), jnp.float32)
            for j in range(NUM_FEAT):
                u = plsc.load_gather(urows_v, [row_ids, uq + j])
                v = plsc.load_gather(irows_v, [row_ids, iq + j])
                acc = acc + u * v
            pr = 1.0 / (1.0 + jnp.exp(-acc))
            plsc.store_scatter(
                out_v, [p * PASS_ELEMS + g * L + lax.iota(jnp.int32, L)], pr)
            return 0

        lax.fori_loop(0, PASS_ELEMS // L, group, 0)

    pltpu.sync_copy(out_v, out_hbm.at[pl.ds(base, B_PER_W)])


@functools.cache
def _build():
    mesh = plsc.VectorSubcoreMesh(core_axis_name="c", subcore_axis_name="s")
    return pl.kernel(
        _body,
        mesh=mesh,
        compiler_params=pltpu.CompilerParams(use_tc_tiling_on_sc=False,
                                             needs_layout_passes=False),
        out_type=jax.ShapeDtypeStruct((BATCH,), jnp.float32),
        scratch_types=[
            pltpu.VMEM((B_PER_W,), jnp.int32),
            pltpu.VMEM((B_PER_W,), jnp.int32),
            pltpu.VMEM((NCHUNK, CHUNK), jnp.int32),
            pltpu.VMEM((NCHUNK, CHUNK), jnp.int32),
            pltpu.VMEM((PASS_ELEMS, 4 * NUM_FEAT), jnp.float32),
            pltpu.VMEM((PASS_ELEMS, 4 * NUM_FEAT), jnp.float32),
            pltpu.VMEM((B_PER_W,), jnp.float32),
            pltpu.SemaphoreType.DMA,
        ],
    )


BC = 2048
SUB = BC // PACK


def _detile_body(in_ref, out_ref):
    x = in_ref[...]
    z = x.reshape(NUM_FEAT, PACK, SUB)
    stacked = z.transpose(1, 0, 2).reshape(PACK * NUM_FEAT, SUB)
    out_ref[...] = jnp.transpose(stacked)


@functools.cache
def _build_detile(n_rows):
    grid = (n_rows + BC - 1) // BC
    return pl.pallas_call(
        _detile_body,
        grid=(grid,),
        in_specs=[pl.BlockSpec((NUM_FEAT, BC), lambda k: (0, k))],
        out_specs=pl.BlockSpec((SUB, PACK * NUM_FEAT), lambda k: (k, 0)),
        out_shape=jax.ShapeDtypeStruct(
            (grid * SUB, PACK * NUM_FEAT), jnp.float32),
    )


def kernel(user_indices, item_indices, w_User, w_Item):
    wU2 = _build_detile(w_User.shape[0])(w_User.T)
    wI2 = _build_detile(w_Item.shape[0])(w_Item.T)
    return _build()(user_indices.astype(jnp.int32),
                    item_indices.astype(jnp.int32),
                    wU2, wI2)


# speedup vs baseline: 2.8665x; 2.0457x over previous
import functools

import jax
import jax.numpy as jnp
from jax import lax
from jax.experimental import pallas as pl
from jax.experimental.pallas import tpu as pltpu
from jax.experimental.pallas import tpu_sc as plsc

BATCH = 16384
NUM_FEAT = 32
L = 16
PACK = 128 // NUM_FEAT

_info = plsc.get_sparse_core_info()
NC, NS = _info.num_cores, _info.num_subcores
NW = NC * NS
B_PER_W = BATCH // NW
CHUNK = 128
NCHUNK = B_PER_W // CHUNK
PASS_ELEMS = 256
NPASS = B_PER_W // PASS_ELEMS
CPP = PASS_ELEMS // CHUNK


def _body(uidx_hbm, iidx_hbm, wU_hbm, wI_hbm, out_hbm,
          uraw_v, iraw_v, ushift_v, ishift_v, urows_v, irows_v, out_v, sem):
    wid = lax.axis_index("s") * NC + lax.axis_index("c")
    base = wid * B_PER_W

    pltpu.sync_copy(uidx_hbm.at[pl.ds(base, B_PER_W)], uraw_v)
    pltpu.sync_copy(iidx_hbm.at[pl.ds(base, B_PER_W)], iraw_v)

    for k in range(NCHUNK):
        for t in range(CHUNK // L):
            o = k * CHUNK + t * L
            u = uraw_v[pl.ds(o, L)]
            i = iraw_v[pl.ds(o, L)]
            ushift_v[k, pl.ds(t * L, L)] = (
                ((u >> LOG_BC) << LOG_SUB) + (u & (SUB - 1)))
            ishift_v[k, pl.ds(t * L, L)] = (
                ((i >> LOG_BC) << LOG_SUB) + (i & (SUB - 1)))

    for p in range(NPASS):
        copies = []
        for c in range(CPP):
            k = p * CPP + c
            copies.append(pltpu.async_copy(
                wU_hbm.at[ushift_v.at[k]],
                urows_v.at[pl.ds(c * CHUNK, CHUNK)], sem))
            copies.append(pltpu.async_copy(
                wI_hbm.at[ishift_v.at[k]],
                irows_v.at[pl.ds(c * CHUNK, CHUNK)], sem))
        for c in copies:
            c.wait()

        def group(g, _):
            row_ids = g * L + lax.iota(jnp.int32, L)
            u16 = uraw_v[pl.ds(p * PASS_ELEMS + g * L, L)]
            i16 = iraw_v[pl.ds(p * PASS_ELEMS + g * L, L)]
            uq = ((u16 >> LOG_SUB) & (PACK - 1)) * NUM_FEAT
            iq = ((i16 >> LOG_SUB) & (PACK - 1)) * NUM_FEAT
            acc = jnp.zeros((L,), jnp.float32)
            for j in range(NUM_FEAT):
                u = plsc.load_gather(urows_v, [row_ids, uq + j])
                v = plsc.load_gather(irows_v, [row_ids, iq + j])
                acc = acc + u * v
            pr = 1.0 / (1.0 + jnp.exp(-acc))
            plsc.store_scatter(
                out_v, [p * PASS_ELEMS + g * L + lax.iota(jnp.int32, L)], pr)
            return 0

        lax.fori_loop(0, PASS_ELEMS // L, group, 0)

    pltpu.sync_copy(out_v, out_hbm.at[pl.ds(base, B_PER_W)])


@functools.cache
def _build():
    mesh = plsc.VectorSubcoreMesh(core_axis_name="c", subcore_axis_name="s")
    return pl.kernel(
        _body,
        mesh=mesh,
        compiler_params=pltpu.CompilerParams(use_tc_tiling_on_sc=False,
                                             needs_layout_passes=False),
        out_type=jax.ShapeDtypeStruct((BATCH,), jnp.float32),
        scratch_types=[
            pltpu.VMEM((B_PER_W,), jnp.int32),
            pltpu.VMEM((B_PER_W,), jnp.int32),
            pltpu.VMEM((NCHUNK, CHUNK), jnp.int32),
            pltpu.VMEM((NCHUNK, CHUNK), jnp.int32),
            pltpu.VMEM((PASS_ELEMS, 4 * NUM_FEAT), jnp.float32),
            pltpu.VMEM((PASS_ELEMS, 4 * NUM_FEAT), jnp.float32),
            pltpu.VMEM((B_PER_W,), jnp.float32),
            pltpu.SemaphoreType.DMA,
        ],
    )


BC = 8192
SUB = BC // PACK
LOG_BC = BC.bit_length() - 1
LOG_SUB = SUB.bit_length() - 1


def _detile_body(in_ref, out_ref):
    x = in_ref[...]
    z = x.reshape(NUM_FEAT, PACK, SUB)
    stacked = z.transpose(1, 0, 2).reshape(PACK * NUM_FEAT, SUB)
    out_ref[...] = jnp.transpose(stacked)


@functools.cache
def _build_detile(n_rows):
    grid = (n_rows + BC - 1) // BC
    return pl.pallas_call(
        _detile_body,
        grid=(grid,),
        in_specs=[pl.BlockSpec((NUM_FEAT, BC), lambda k: (0, k))],
        out_specs=pl.BlockSpec((SUB, PACK * NUM_FEAT), lambda k: (k, 0)),
        out_shape=jax.ShapeDtypeStruct(
            (grid * SUB, PACK * NUM_FEAT), jnp.float32),
    )


def kernel(user_indices, item_indices, w_User, w_Item):
    wU2 = _build_detile(w_User.shape[0])(w_User.T)
    wI2 = _build_detile(w_Item.shape[0])(w_Item.T)
    return _build()(user_indices.astype(jnp.int32),
                    item_indices.astype(jnp.int32),
                    wU2, wI2)


# speedup vs baseline: 4.0064x; 1.3977x over previous
import functools

import jax
import jax.numpy as jnp
from jax import lax
from jax.experimental import pallas as pl
from jax.experimental.pallas import tpu as pltpu
from jax.experimental.pallas import tpu_sc as plsc

BATCH = 16384
NUM_FEAT = 32
L = 16
PACK = 128 // NUM_FEAT

_info = plsc.get_sparse_core_info()
NC, NS = _info.num_cores, _info.num_subcores
NW = NC * NS
B_PER_W = BATCH // NW
CHUNK = 128
NCHUNK = B_PER_W // CHUNK
PASS_ELEMS = 256
NPASS = B_PER_W // PASS_ELEMS
CPP = PASS_ELEMS // CHUNK


def _body(uidx_hbm, iidx_hbm, wU_hbm, wI_hbm, out_hbm,
          uraw_v, iraw_v, ushift_v, ishift_v, urows_v, irows_v, out_v, sem):
    wid = lax.axis_index("s") * NC + lax.axis_index("c")
    base = wid * B_PER_W

    pltpu.sync_copy(uidx_hbm.at[pl.ds(base, B_PER_W)], uraw_v)
    pltpu.sync_copy(iidx_hbm.at[pl.ds(base, B_PER_W)], iraw_v)

    for k in range(NCHUNK):
        for t in range(CHUNK // L):
            o = k * CHUNK + t * L
            u = uraw_v[pl.ds(o, L)]
            i = iraw_v[pl.ds(o, L)]
            ushift_v[k, pl.ds(t * L, L)] = (
                ((u >> LOG_BC) << LOG_SUB) + (u & (SUB - 1)))
            ishift_v[k, pl.ds(t * L, L)] = (
                ((i >> LOG_BC) << LOG_SUB) + (i & (SUB - 1)))

    for p in range(NPASS):
        copies = []
        for c in range(CPP):
            k = p * CPP + c
            copies.append(pltpu.async_copy(
                wU_hbm.at[ushift_v.at[k]],
                urows_v.at[pl.ds(c * CHUNK, CHUNK)], sem))
            copies.append(pltpu.async_copy(
                wI_hbm.at[ishift_v.at[k]],
                irows_v.at[pl.ds(c * CHUNK, CHUNK)], sem))
        for c in copies:
            c.wait()

        def group(g, _):
            row_ids = g * L + lax.iota(jnp.int32, L)
            u16 = uraw_v[pl.ds(p * PASS_ELEMS + g * L, L)]
            i16 = iraw_v[pl.ds(p * PASS_ELEMS + g * L, L)]
            uq = ((u16 >> LOG_SUB) & (PACK - 1)) * NUM_FEAT
            iq = ((i16 >> LOG_SUB) & (PACK - 1)) * NUM_FEAT
            acc = jnp.zeros((L,), jnp.float32)
            for j in range(NUM_FEAT):
                u = plsc.load_gather(urows_v, [row_ids, uq + j])
                v = plsc.load_gather(irows_v, [row_ids, iq + j])
                acc = acc + u * v
            pr = 1.0 / (1.0 + jnp.exp(-acc))
            plsc.store_scatter(
                out_v, [p * PASS_ELEMS + g * L + lax.iota(jnp.int32, L)], pr)
            return 0

        lax.fori_loop(0, PASS_ELEMS // L, group, 0)

    pltpu.sync_copy(out_v, out_hbm.at[pl.ds(base, B_PER_W)])


@functools.cache
def _build():
    mesh = plsc.VectorSubcoreMesh(core_axis_name="c", subcore_axis_name="s")
    return pl.kernel(
        _body,
        mesh=mesh,
        compiler_params=pltpu.CompilerParams(use_tc_tiling_on_sc=False,
                                             needs_layout_passes=False),
        out_type=jax.ShapeDtypeStruct((BATCH,), jnp.float32),
        scratch_types=[
            pltpu.VMEM((B_PER_W,), jnp.int32),
            pltpu.VMEM((B_PER_W,), jnp.int32),
            pltpu.VMEM((NCHUNK, CHUNK), jnp.int32),
            pltpu.VMEM((NCHUNK, CHUNK), jnp.int32),
            pltpu.VMEM((PASS_ELEMS, 4 * NUM_FEAT), jnp.float32),
            pltpu.VMEM((PASS_ELEMS, 4 * NUM_FEAT), jnp.float32),
            pltpu.VMEM((B_PER_W,), jnp.float32),
            pltpu.SemaphoreType.DMA,
        ],
    )


BC = 32768
SUB = BC // PACK
LOG_BC = BC.bit_length() - 1
LOG_SUB = SUB.bit_length() - 1


def _detile_body(in_ref, out_ref):
    x = in_ref[...]
    z = x.reshape(NUM_FEAT, PACK, SUB)
    stacked = z.transpose(1, 0, 2).reshape(PACK * NUM_FEAT, SUB)
    out_ref[...] = jnp.transpose(stacked)


@functools.cache
def _build_detile(n_rows):
    grid = (n_rows + BC - 1) // BC
    return pl.pallas_call(
        _detile_body,
        grid=(grid,),
        in_specs=[pl.BlockSpec((NUM_FEAT, BC), lambda k: (0, k))],
        out_specs=pl.BlockSpec((SUB, PACK * NUM_FEAT), lambda k: (k, 0)),
        out_shape=jax.ShapeDtypeStruct(
            (grid * SUB, PACK * NUM_FEAT), jnp.float32),
    )


def kernel(user_indices, item_indices, w_User, w_Item):
    wU2 = _build_detile(w_User.shape[0])(w_User.T)
    wI2 = _build_detile(w_Item.shape[0])(w_Item.T)
    return _build()(user_indices.astype(jnp.int32),
                    item_indices.astype(jnp.int32),
                    wU2, wI2)


# speedup vs baseline: 4.0627x; 1.0141x over previous
import functools

import jax
import jax.numpy as jnp
from jax import lax
from jax.experimental import pallas as pl
from jax.experimental.pallas import tpu as pltpu
from jax.experimental.pallas import tpu_sc as plsc

BATCH = 16384
NUM_FEAT = 32
L = 16
PACK = 128 // NUM_FEAT

_info = plsc.get_sparse_core_info()
NC, NS = _info.num_cores, _info.num_subcores
NW = NC * NS
B_PER_W = BATCH // NW
CHUNK = 128
NCHUNK = B_PER_W // CHUNK
PASS_ELEMS = 256
NPASS = B_PER_W // PASS_ELEMS
CPP = PASS_ELEMS // CHUNK


def _body(uidx_hbm, iidx_hbm, wU_hbm, wI_hbm, out_hbm,
          uraw_v, iraw_v, ushift_v, ishift_v, urows_v, irows_v, out_v, sem):
    wid = lax.axis_index("s") * NC + lax.axis_index("c")
    base = wid * B_PER_W

    pltpu.sync_copy(uidx_hbm.at[pl.ds(base, B_PER_W)], uraw_v)
    pltpu.sync_copy(iidx_hbm.at[pl.ds(base, B_PER_W)], iraw_v)

    for k in range(NCHUNK):
        for t in range(CHUNK // L):
            o = k * CHUNK + t * L
            u = uraw_v[pl.ds(o, L)]
            i = iraw_v[pl.ds(o, L)]
            ushift_v[k, pl.ds(t * L, L)] = (
                ((u >> LOG_BC) << LOG_SUB) + (u & (SUB - 1)))
            ishift_v[k, pl.ds(t * L, L)] = (
                ((i >> LOG_BC) << LOG_SUB) + (i & (SUB - 1)))

    for p in range(NPASS):
        copies = []
        for c in range(CPP):
            k = p * CPP + c
            copies.append(pltpu.async_copy(
                wU_hbm.at[ushift_v.at[k]],
                urows_v.at[pl.ds(c * CHUNK, CHUNK)], sem))
            copies.append(pltpu.async_copy(
                wI_hbm.at[ishift_v.at[k]],
                irows_v.at[pl.ds(c * CHUNK, CHUNK)], sem))
        for c in copies:
            c.wait()

        def group(g, _):
            row_ids = g * L + lax.iota(jnp.int32, L)
            u16 = uraw_v[pl.ds(p * PASS_ELEMS + g * L, L)]
            i16 = iraw_v[pl.ds(p * PASS_ELEMS + g * L, L)]
            uq = ((u16 >> LOG_SUB) & (PACK - 1)) * NUM_FEAT
            iq = ((i16 >> LOG_SUB) & (PACK - 1)) * NUM_FEAT
            acc = jnp.zeros((L,), jnp.float32)
            for j in range(NUM_FEAT):
                u = plsc.load_gather(urows_v, [row_ids, uq + j])
                v = plsc.load_gather(irows_v, [row_ids, iq + j])
                acc = acc + u * v
            pr = 1.0 / (1.0 + jnp.exp(-acc))
            plsc.store_scatter(
                out_v, [p * PASS_ELEMS + g * L + lax.iota(jnp.int32, L)], pr)
            return 0

        lax.fori_loop(0, PASS_ELEMS // L, group, 0)

    pltpu.sync_copy(out_v, out_hbm.at[pl.ds(base, B_PER_W)])


@functools.cache
def _build():
    mesh = plsc.VectorSubcoreMesh(core_axis_name="c", subcore_axis_name="s")
    return pl.kernel(
        _body,
        mesh=mesh,
        compiler_params=pltpu.CompilerParams(use_tc_tiling_on_sc=False,
                                             needs_layout_passes=False),
        out_type=jax.ShapeDtypeStruct((BATCH,), jnp.float32),
        scratch_types=[
            pltpu.VMEM((B_PER_W,), jnp.int32),
            pltpu.VMEM((B_PER_W,), jnp.int32),
            pltpu.VMEM((NCHUNK, CHUNK), jnp.int32),
            pltpu.VMEM((NCHUNK, CHUNK), jnp.int32),
            pltpu.VMEM((PASS_ELEMS, 4 * NUM_FEAT), jnp.float32),
            pltpu.VMEM((PASS_ELEMS, 4 * NUM_FEAT), jnp.float32),
            pltpu.VMEM((B_PER_W,), jnp.float32),
            pltpu.SemaphoreType.DMA,
        ],
    )


BC = 65536
SUB = BC // PACK
LOG_BC = BC.bit_length() - 1
LOG_SUB = SUB.bit_length() - 1


def _detile_body(in_ref, out_ref):
    x = in_ref[...]
    z = x.reshape(NUM_FEAT, PACK, SUB)
    stacked = z.transpose(1, 0, 2).reshape(PACK * NUM_FEAT, SUB)
    out_ref[...] = jnp.transpose(stacked)


@functools.cache
def _build_detile(n_rows):
    grid = (n_rows + BC - 1) // BC
    return pl.pallas_call(
        _detile_body,
        grid=(grid,),
        in_specs=[pl.BlockSpec((NUM_FEAT, BC), lambda k: (0, k))],
        out_specs=pl.BlockSpec((SUB, PACK * NUM_FEAT), lambda k: (k, 0)),
        out_shape=jax.ShapeDtypeStruct(
            (grid * SUB, PACK * NUM_FEAT), jnp.float32),
    )


def kernel(user_indices, item_indices, w_User, w_Item):
    wU2 = _build_detile(w_User.shape[0])(w_User.T)
    wI2 = _build_detile(w_Item.shape[0])(w_Item.T)
    return _build()(user_indices.astype(jnp.int32),
                    item_indices.astype(jnp.int32),
                    wU2, wI2)
